# baseline (speedup 1.0000x reference)
import functools

import jax
import jax.numpy as jnp
from jax import lax
from jax.experimental import pallas as pl
from jax.experimental.pallas import tpu as pltpu
from jax.experimental.pallas import tpu_sc as plsc

_B = 16384
_D = 32

_info = plsc.get_sparse_core_info()
_NC = _info.num_cores
_NS = _info.num_subcores
_NW = _NC * _NS
_B_PER_W = _B // _NW
_CHUNK = 128
_NCHUNK = _B_PER_W // _CHUNK


def _make_gather():
    mesh = plsc.VectorSubcoreMesh(core_axis_name="c", subcore_axis_name="s")

    @functools.partial(
        pl.kernel,
        mesh=mesh,
        out_type=jax.ShapeDtypeStruct((_B, _D), jnp.float32),
        compiler_params=pltpu.CompilerParams(use_tc_tiling_on_sc=False),
        scratch_types=[
            pltpu.VMEM((_NCHUNK, _CHUNK), jnp.int32),
            pltpu.VMEM((_B_PER_W, _D), jnp.float32),
            pltpu.SemaphoreType.DMA,
        ],
    )
    def gather_kernel(idx_hbm, table_hbm, out_hbm, idx_v, rows_v, sem):
        wid = lax.axis_index("s") * _NC + lax.axis_index("c")
        base = wid * _B_PER_W
        pltpu.sync_copy(idx_hbm.at[wid], idx_v)
        copies = [
            pltpu.async_copy(
                table_hbm.at[idx_v.at[j]],
                rows_v.at[pl.ds(j * _CHUNK, _CHUNK)],
                sem,
            )
            for j in range(_NCHUNK)
        ]
        for c in copies:
            c.wait()
        pltpu.sync_copy(rows_v, out_hbm.at[pl.ds(base, _B_PER_W)])

    return gather_kernel


_gather = _make_gather()


def kernel(color_idx, table):
    idx = color_idx.astype(jnp.int32).reshape(_NW, _NCHUNK, _CHUNK)
    return _gather(idx, table)


# speedup vs baseline: 3.8099x; 3.8099x over previous
import functools

import jax
import jax.numpy as jnp
from jax import lax
from jax.experimental import pallas as pl
from jax.experimental.pallas import tpu as pltpu
from jax.experimental.pallas import tpu_sc as plsc

_B = 16384
_D = 32

_info = plsc.get_sparse_core_info()
_NC = _info.num_cores
_NS = _info.num_subcores
_NW = _NC * _NS
_B_PER_W = _B // _NW
_LANES = 16
_GH = 8
_NG = _B_PER_W // _GH


def _make_gather():
    mesh = plsc.VectorSubcoreMesh(core_axis_name="c", subcore_axis_name="s")

    @functools.partial(
        pl.kernel,
        mesh=mesh,
        out_type=jax.ShapeDtypeStruct((_D, _B), jnp.float32),
        compiler_params=pltpu.CompilerParams(needs_layout_passes=False),
        scratch_types=[
            pltpu.VMEM((_B_PER_W + _LANES,), jnp.int32),
            pltpu.VMEM((2, _GH, _D, 128), jnp.float32),
            pltpu.VMEM((_D * _B_PER_W,), jnp.float32),
            pltpu.SemaphoreType.DMA,
            pltpu.SemaphoreType.DMA,
            pltpu.SemaphoreType.DMA,
        ],
    )
    def gather_kernel(idx_hbm, tab_t_hbm, out_t_hbm, idx_v, blk_v, cols_v,
                      sem0, sem1, osem):
        wid = lax.axis_index("s") * _NC + lax.axis_index("c")
        base = wid * _B_PER_W
        sems = [sem0, sem1]
        pltpu.sync_copy(idx_hbm.at[pl.ds(base, _B_PER_W)],
                        idx_v.at[pl.ds(0, _B_PER_W)])

        row16a = lax.iota(jnp.int32, _LANES)
        row16b = row16a + _LANES
        scat_a = row16a * _B_PER_W
        scat_b = row16b * _B_PER_W

        def fire(g, bank):
            vec = idx_v[pl.ds(g * _GH, _LANES)]
            for k in range(_GH):
                c = vec[k]
                q = pl.multiple_of((c >> 7) * 128, 128)
                pltpu.async_copy(
                    tab_t_hbm.at[:, pl.ds(q, 128)],
                    blk_v.at[bank, k],
                    sems[bank],
                )

        def drain_and_extract(g, bank):
            for k in range(_GH):
                pltpu.make_async_copy(
                    tab_t_hbm.at[:, pl.ds(0, 128)],
                    blk_v.at[bank, k],
                    sems[bank],
                ).wait()
            vec = idx_v[pl.ds(g * _GH, _LANES)] & 127
            for k in range(_GH):
                pos = g * _GH + k
                col_vec = jnp.full((_LANES,), vec[k], dtype=jnp.int32)
                va = plsc.load_gather(blk_v.at[bank, k], [row16a, col_vec])
                vb = plsc.load_gather(blk_v.at[bank, k], [row16b, col_vec])
                plsc.store_scatter(cols_v, [scat_a + pos], va)
                plsc.store_scatter(cols_v, [scat_b + pos], vb)

        fire(0, 0)

        def body(g):
            bank = lax.rem(g, 2)
            pl.when(bank == 0)(lambda: fire(g, 0))
            pl.when(bank == 1)(lambda: fire(g, 1))
            pl.when(bank == 0)(lambda: drain_and_extract(g - 1, 1))
            pl.when(bank == 1)(lambda: drain_and_extract(g - 1, 0))

        pl.loop(1, _NG)(body)
        drain_and_extract(_NG - 1, (_NG - 1) % 2)

        for d in range(_D):
            pltpu.async_copy(
                cols_v.at[pl.ds(d * _B_PER_W, _B_PER_W)],
                out_t_hbm.at[d, pl.ds(base, _B_PER_W)],
                osem,
            )
        for d in range(_D):
            pltpu.make_async_copy(
                cols_v.at[pl.ds(d * _B_PER_W, _B_PER_W)],
                out_t_hbm.at[d, pl.ds(base, _B_PER_W)],
                osem,
            ).wait()

    return gather_kernel


_gather = _make_gather()


def kernel(color_idx, table):
    idx = color_idx.astype(jnp.int32)
    out_t = _gather(idx, table.T)
    return out_t.T


# speedup vs baseline: 4.1440x; 1.0877x over previous
import functools

import jax
import jax.numpy as jnp
from jax import lax
from jax.experimental import pallas as pl
from jax.experimental.pallas import tpu as pltpu
from jax.experimental.pallas import tpu_sc as plsc

_B = 16384
_D = 32

_info = plsc.get_sparse_core_info()
_NC = _info.num_cores
_NS = _info.num_subcores
_NW = _NC * _NS
_B_PER_W = _B // _NW
_LANES = 16
_GH = 8
_NG = _B_PER_W // _GH


def _make_gather():
    mesh = plsc.VectorSubcoreMesh(core_axis_name="c", subcore_axis_name="s")

    @functools.partial(
        pl.kernel,
        mesh=mesh,
        out_type=jax.ShapeDtypeStruct((_D, _B), jnp.float32),
        compiler_params=pltpu.CompilerParams(needs_layout_passes=False),
        scratch_types=[
            pltpu.VMEM((_B_PER_W + _LANES,), jnp.int32),
            pltpu.VMEM((3, _GH, _D, 128), jnp.float32),
            pltpu.VMEM((_D * _B_PER_W,), jnp.float32),
            pltpu.SemaphoreType.DMA,
            pltpu.SemaphoreType.DMA,
            pltpu.SemaphoreType.DMA,
            pltpu.SemaphoreType.DMA,
        ],
    )
    def gather_kernel(idx_hbm, tab_t_hbm, out_t_hbm, idx_v, blk_v, cols_v,
                      sem0, sem1, sem2, osem):
        wid = lax.axis_index("s") * _NC + lax.axis_index("c")
        base = wid * _B_PER_W
        sems = [sem0, sem1, sem2]
        pltpu.sync_copy(idx_hbm.at[pl.ds(base, _B_PER_W)],
                        idx_v.at[pl.ds(0, _B_PER_W)])

        row16a = lax.iota(jnp.int32, _LANES)
        row16b = row16a + _LANES
        scat_a = row16a * _B_PER_W
        scat_b = row16b * _B_PER_W

        def fire(g, bank):
            vec = idx_v[pl.ds(g * _GH, _LANES)]
            for k in range(_GH):
                c = vec[k]
                q = pl.multiple_of(c & -128, 128)
                pltpu.async_copy(
                    tab_t_hbm.at[:, pl.ds(q, 128)],
                    blk_v.at[bank, k],
                    sems[bank],
                )

        def drain_and_extract(g, bank):
            for k in range(_GH):
                pltpu.make_async_copy(
                    tab_t_hbm.at[:, pl.ds(0, 128)],
                    blk_v.at[bank, k],
                    sems[bank],
                ).wait()
            vec = idx_v[pl.ds(g * _GH, _LANES)] & 127
            for k in range(_GH):
                pos = g * _GH + k
                col_vec = jnp.full((_LANES,), vec[k], dtype=jnp.int32)
                va = plsc.load_gather(blk_v.at[bank, k], [row16a, col_vec])
                vb = plsc.load_gather(blk_v.at[bank, k], [row16b, col_vec])
                plsc.store_scatter(cols_v, [scat_a + pos], va)
                plsc.store_scatter(cols_v, [scat_b + pos], vb)

        fire(0, 0)
        fire(1, 1)

        def body(g):
            bank = lax.rem(g, 3)
            pl.when(bank == 0)(lambda: fire(g, 0))
            pl.when(bank == 1)(lambda: fire(g, 1))
            pl.when(bank == 2)(lambda: fire(g, 2))
            old = lax.rem(g + 1, 3)
            pl.when(old == 0)(lambda: drain_and_extract(g - 2, 0))
            pl.when(old == 1)(lambda: drain_and_extract(g - 2, 1))
            pl.when(old == 2)(lambda: drain_and_extract(g - 2, 2))

        pl.loop(2, _NG)(body)
        drain_and_extract(_NG - 2, (_NG - 2) % 3)
        drain_and_extract(_NG - 1, (_NG - 1) % 3)

        for d in range(_D):
            pltpu.async_copy(
                cols_v.at[pl.ds(d * _B_PER_W, _B_PER_W)],
                out_t_hbm.at[d, pl.ds(base, _B_PER_W)],
                osem,
            )
        for d in range(_D):
            pltpu.make_async_copy(
                cols_v.at[pl.ds(d * _B_PER_W, _B_PER_W)],
                out_t_hbm.at[d, pl.ds(base, _B_PER_W)],
                osem,
            ).wait()

    return gather_kernel


_gather = _make_gather()


def kernel(color_idx, table):
    idx = color_idx.astype(jnp.int32)
    out_t = _gather(idx, table.T)
    return out_t.T
